# baseline (speedup 1.0000x reference)
import dataclasses
import functools

import jax
import jax.numpy as jnp
from jax import lax
from jax.experimental import pallas as pl
from jax.experimental.pallas import tpu as pltpu
from jax.experimental.pallas import tpu_sc as plsc

B = 16384
D = 64
NC = 2
NS = 16
NW = NC * NS
ROWS = B // NW
GW = 128
NG = ROWS // GW
L = 16
DCH = D // L


def _body(uemb, pemb, cemb, ubias, pbias, gbias, uidx, pidx, cidx,
          out, uix_v, pix_v, cix_v, u_v, p_v, c_v, ub_v, pb_v, g_v, o_v, sem):
    cid = lax.axis_index("c")
    sid = lax.axis_index("s")
    wid = sid * NC + cid
    base = wid * ROWS

    pltpu.sync_copy(uidx.at[wid], uix_v)
    pltpu.sync_copy(pidx.at[wid], pix_v)
    pltpu.sync_copy(cidx.at[wid], cix_v)
    pltpu.sync_copy(gbias, g_v.at[pl.ds(0, 1)])

    cps = []
    for j in range(NG):
        dst = pl.ds(j * GW, GW)
        cps.append(pltpu.async_copy(uemb.at[uix_v.at[j]], u_v.at[dst], sem))
        cps.append(pltpu.async_copy(pemb.at[pix_v.at[j]], p_v.at[dst], sem))
        cps.append(pltpu.async_copy(cemb.at[cix_v.at[j]], c_v.at[dst], sem))
        cps.append(pltpu.async_copy(ubias.at[uix_v.at[j]], ub_v.at[dst], sem))
        cps.append(pltpu.async_copy(pbias.at[pix_v.at[j]], pb_v.at[dst], sem))
    for cp in cps:
        cp.wait()

    last_lane = lax.iota(jnp.int32, L) == (L - 1)

    @pl.loop(0, ROWS)
    def _(b):
        acc = None
        for k in range(DCH):
            sl = pl.ds(k * L, L)
            t = u_v[b, sl] * (p_v[b, sl] + jnp.float32(0.3) * c_v[b, sl])
            acc = t if acc is None else acc + t
        s = plsc.cumsum(acc)
        bvec = jnp.full((L,), b, jnp.int32)
        plsc.store_scatter(o_v, [bvec], s, mask=last_lane)

    g = g_v[pl.ds(0, L)][0]
    @pl.loop(0, ROWS // L)
    def _(i):
        sl = pl.ds(i * L, L)
        o_v[sl] = o_v[sl] + ub_v[sl] + pb_v[sl] + g

    pltpu.sync_copy(o_v, out.at[pl.ds(base, ROWS)])


@functools.partial(jax.jit, static_argnames=())
def kernel(user_idx, product_idx, category_idx, user_emb, product_emb,
           category_emb, user_bias, product_bias, global_bias):
    mesh = plsc.VectorSubcoreMesh(core_axis_name="c", subcore_axis_name="s",
                                  num_cores=NC, num_subcores=NS)
    cp = pltpu.CompilerParams(needs_layout_passes=False,
                              use_tc_tiling_on_sc=False)
    sc_call = pl.kernel(
        _body,
        out_type=jax.ShapeDtypeStruct((B,), jnp.float32),
        mesh=mesh,
        compiler_params=cp,
        scratch_types=[
            pltpu.VMEM((NG, GW), jnp.int32),
            pltpu.VMEM((NG, GW), jnp.int32),
            pltpu.VMEM((NG, GW), jnp.int32),
            pltpu.VMEM((ROWS, D), jnp.float32),
            pltpu.VMEM((ROWS, D), jnp.float32),
            pltpu.VMEM((ROWS, D), jnp.float32),
            pltpu.VMEM((ROWS,), jnp.float32),
            pltpu.VMEM((ROWS,), jnp.float32),
            pltpu.VMEM((L,), jnp.float32),
            pltpu.VMEM((ROWS,), jnp.float32),
            pltpu.SemaphoreType.DMA,
        ],
    )
    return sc_call(
        user_emb,
        product_emb,
        category_emb,
        user_bias.reshape(-1),
        product_bias.reshape(-1),
        global_bias,
        user_idx.reshape(NW, NG, GW).astype(jnp.int32),
        product_idx.reshape(NW, NG, GW).astype(jnp.int32),
        category_idx.reshape(NW, NG, GW).astype(jnp.int32),
    )


# speedup vs baseline: 1.4602x; 1.4602x over previous
import functools

import jax
import jax.numpy as jnp
from jax import lax
from jax.experimental import pallas as pl
from jax.experimental.pallas import tpu as pltpu
from jax.experimental.pallas import tpu_sc as plsc

B = 16384
D = 64
NC = 2
NS = 16
NW = NC * NS
ROWS = B // NW
L = 16
DCH = D // L
GW = 128
NG = ROWS // GW


def _pc_body(pemb, cemb, ubias, pbias, gbias, uidx, pidx, cidx,
             comb_out, bsum_out, uix_v, pix_v, cix_v, p_v, c_v,
             ub_v, pb_v, g_v, bs_v, sem):
    cid = lax.axis_index("c")
    sid = lax.axis_index("s")
    wid = sid * NC + cid
    base = wid * ROWS

    pltpu.sync_copy(uidx.at[wid], uix_v)
    pltpu.sync_copy(pidx.at[wid], pix_v)
    pltpu.sync_copy(cidx.at[wid], cix_v)
    pltpu.sync_copy(gbias, g_v.at[pl.ds(0, 1)])

    cps = []
    for j in range(NG):
        dst = pl.ds(j * GW, GW)
        cps.append(pltpu.async_copy(pemb.at[pix_v.at[j]], p_v.at[dst], sem))
        cps.append(pltpu.async_copy(cemb.at[cix_v.at[j]], c_v.at[dst], sem))
        cps.append(pltpu.async_copy(ubias.at[uix_v.at[j]],
                                    ub_v.at[dst], sem))
        cps.append(pltpu.async_copy(pbias.at[pix_v.at[j]],
                                    pb_v.at[dst], sem))
    for cp in cps:
        cp.wait()

    @pl.loop(0, ROWS)
    def _(b):
        for k in range(DCH):
            sl = pl.ds(k * L, L)
            p_v[b, sl] = p_v[b, sl] + jnp.float32(0.3) * c_v[b, sl]

    g = g_v[pl.ds(0, L)][0]

    @pl.loop(0, ROWS // L)
    def _(i):
        sl = pl.ds(i * L, L)
        bs_v[sl] = ub_v[sl] + pb_v[sl] + g

    pltpu.sync_copy(p_v, comb_out.at[pl.ds(base, ROWS)])
    pltpu.sync_copy(bs_v, bsum_out.at[pl.ds(base, ROWS)])


def _u_body(uemb, comb, bsum, uidx, out, uix_v, u_v, cb_v, bs_v, o_v, sem):
    cid = lax.axis_index("c")
    sid = lax.axis_index("s")
    wid = sid * NC + cid
    base = wid * ROWS

    pltpu.sync_copy(uidx.at[pl.ds(base, ROWS)], uix_v)
    pltpu.sync_copy(comb.at[pl.ds(base * D, ROWS * D)], cb_v)
    pltpu.sync_copy(bsum.at[pl.ds(base, ROWS)], bs_v)

    @pl.loop(0, ROWS // L)
    def _(g):
        vu = uix_v[pl.ds(g * L, L)]
        for l in range(L):
            pltpu.async_copy(uemb.at[vu[l]], u_v.at[g * L + l], sem)

    pltpu.make_async_copy(uemb.at[pl.ds(0, ROWS)], u_v, sem).wait()

    last_lane = lax.iota(jnp.int32, L) == (L - 1)

    @pl.loop(0, ROWS)
    def _(b):
        acc = None
        for k in range(DCH):
            sl = pl.ds(k * L, L)
            t = u_v[b, sl] * cb_v[pl.ds(b * D + k * L, L)]
            acc = t if acc is None else acc + t
        s = plsc.cumsum(acc)
        bvec = jnp.full((L,), b, jnp.int32)
        plsc.store_scatter(o_v, [bvec], s, mask=last_lane)

    @pl.loop(0, ROWS // L)
    def _(i):
        sl = pl.ds(i * L, L)
        o_v[sl] = o_v[sl] + bs_v[sl]

    pltpu.sync_copy(o_v, out.at[pl.ds(base, ROWS)])


@functools.partial(jax.jit, static_argnames=())
def kernel(user_idx, product_idx, category_idx, user_emb, product_emb,
           category_emb, user_bias, product_bias, global_bias):
    mesh = plsc.VectorSubcoreMesh(core_axis_name="c", subcore_axis_name="s",
                                  num_cores=NC, num_subcores=NS)
    pc_call = pl.kernel(
        _pc_body,
        out_type=(jax.ShapeDtypeStruct((B, D), jnp.float32),
                  jax.ShapeDtypeStruct((B,), jnp.float32)),
        mesh=mesh,
        compiler_params=pltpu.CompilerParams(needs_layout_passes=False,
                                             use_tc_tiling_on_sc=False),
        scratch_types=[
            pltpu.VMEM((NG, GW), jnp.int32),
            pltpu.VMEM((NG, GW), jnp.int32),
            pltpu.VMEM((NG, GW), jnp.int32),
            pltpu.VMEM((ROWS, D), jnp.float32),
            pltpu.VMEM((ROWS, D), jnp.float32),
            pltpu.VMEM((ROWS,), jnp.float32),
            pltpu.VMEM((ROWS,), jnp.float32),
            pltpu.VMEM((L,), jnp.float32),
            pltpu.VMEM((ROWS,), jnp.float32),
            pltpu.SemaphoreType.DMA,
        ],
    )
    u_call = pl.kernel(
        _u_body,
        out_type=jax.ShapeDtypeStruct((B,), jnp.float32),
        mesh=mesh,
        compiler_params=pltpu.CompilerParams(needs_layout_passes=False,
                                             use_tc_tiling_on_sc=True),
        scratch_types=[
            pltpu.VMEM((ROWS,), jnp.int32),
            pltpu.VMEM((ROWS, D), jnp.float32),
            pltpu.VMEM((ROWS * D,), jnp.float32),
            pltpu.VMEM((ROWS,), jnp.float32),
            pltpu.VMEM((ROWS,), jnp.float32),
            pltpu.SemaphoreType.DMA,
        ],
    )
    idx3 = lambda a: a.reshape(NW, NG, GW)
    comb, bsum = pc_call(product_emb, category_emb,
                         user_bias.reshape(-1), product_bias.reshape(-1),
                         global_bias, idx3(user_idx), idx3(product_idx),
                         idx3(category_idx))
    return u_call(user_emb, comb.reshape(-1), bsum, user_idx)


# speedup vs baseline: 1.4649x; 1.0032x over previous
import functools

import jax
import jax.numpy as jnp
from jax import lax
from jax.experimental import pallas as pl
from jax.experimental.pallas import tpu as pltpu
from jax.experimental.pallas import tpu_sc as plsc

B = 16384
D = 64
NC = 2
NS = 16
NW = NC * NS
ROWS = B // NW
L = 16
DCH = D // L
GW = 128
NG = ROWS // GW


def _pc_body(pemb, cemb, ubias, pbias, gbias, uidx, pidx, cidx,
             comb_out, bsum_out, uix_v, pix_v, cix_v, p_v, c_v,
             ub_v, pb_v, g_v, bs_v, sem):
    cid = lax.axis_index("c")
    sid = lax.axis_index("s")
    wid = sid * NC + cid
    base = wid * ROWS

    pltpu.sync_copy(uidx.at[wid], uix_v)
    pltpu.sync_copy(pidx.at[wid], pix_v)
    pltpu.sync_copy(cidx.at[wid], cix_v)
    pltpu.sync_copy(gbias, g_v.at[pl.ds(0, 1)])

    cps = []
    for j in range(NG):
        dst = pl.ds(j * GW, GW)
        cps.append(pltpu.async_copy(pemb.at[pix_v.at[j]], p_v.at[dst], sem))
        cps.append(pltpu.async_copy(cemb.at[cix_v.at[j]], c_v.at[dst], sem))
        cps.append(pltpu.async_copy(ubias.at[uix_v.at[j]],
                                    ub_v.at[dst], sem))
        cps.append(pltpu.async_copy(pbias.at[pix_v.at[j]],
                                    pb_v.at[dst], sem))
    for cp in cps:
        cp.wait()

    @pl.loop(0, ROWS)
    def _(b):
        for k in range(DCH):
            sl = pl.ds(k * L, L)
            p_v[b, sl] = p_v[b, sl] + jnp.float32(0.3) * c_v[b, sl]

    g = g_v[pl.ds(0, L)][0]

    @pl.loop(0, ROWS // L)
    def _(i):
        sl = pl.ds(i * L, L)
        bs_v[sl] = ub_v[sl] + pb_v[sl] + g

    pltpu.sync_copy(p_v, comb_out.at[pl.ds(base, ROWS)])
    pltpu.sync_copy(bs_v, bsum_out.at[pl.ds(base, ROWS)])


def _u_body(uemb, comb, bsum, uidx, out, uix_v, u_v, cb_v, bs_v, o_v, sem):
    cid = lax.axis_index("c")
    sid = lax.axis_index("s")
    wid = sid * NC + cid
    base = wid * ROWS

    pltpu.sync_copy(uidx.at[pl.ds(base, ROWS)], uix_v)
    pltpu.sync_copy(comb.at[pl.ds(base * D, ROWS * D)], cb_v)
    pltpu.sync_copy(bsum.at[pl.ds(base, ROWS)], bs_v)

    @pl.loop(0, ROWS // L)
    def _(g):
        vu = uix_v[pl.ds(g * L, L)]
        for l in range(L):
            pltpu.async_copy(uemb.at[vu[l]], u_v.at[g * L + l], sem)

    pltpu.make_async_copy(uemb.at[pl.ds(0, ROWS)], u_v, sem).wait()

    last_lane = lax.iota(jnp.int32, L) == (L - 1)

    @pl.loop(0, ROWS)
    def _(b):
        acc = None
        for k in range(DCH):
            sl = pl.ds(k * L, L)
            t = u_v[b, sl] * cb_v[pl.ds(b * D + k * L, L)]
            acc = t if acc is None else acc + t
        s = plsc.cumsum(acc)
        bvec = jnp.full((L,), b, jnp.int32)
        plsc.store_scatter(o_v, [bvec], s, mask=last_lane)

    @pl.loop(0, ROWS // L)
    def _(i):
        sl = pl.ds(i * L, L)
        o_v[sl] = o_v[sl] + bs_v[sl]

    pltpu.sync_copy(o_v, out.at[pl.ds(base, ROWS)])


@functools.partial(jax.jit, static_argnames=())
def kernel(user_idx, product_idx, category_idx, user_emb, product_emb,
           category_emb, user_bias, product_bias, global_bias):
    mesh = plsc.VectorSubcoreMesh(core_axis_name="c", subcore_axis_name="s",
                                  num_cores=NC, num_subcores=NS)
    pc_call = pl.kernel(
        _pc_body,
        out_type=(jax.ShapeDtypeStruct((B, D), jnp.float32),
                  jax.ShapeDtypeStruct((B,), jnp.float32)),
        mesh=mesh,
        compiler_params=pltpu.CompilerParams(needs_layout_passes=False,
                                             use_tc_tiling_on_sc=False,
                                             skip_device_barrier=True),
        scratch_types=[
            pltpu.VMEM((NG, GW), jnp.int32),
            pltpu.VMEM((NG, GW), jnp.int32),
            pltpu.VMEM((NG, GW), jnp.int32),
            pltpu.VMEM((ROWS, D), jnp.float32),
            pltpu.VMEM((ROWS, D), jnp.float32),
            pltpu.VMEM((ROWS,), jnp.float32),
            pltpu.VMEM((ROWS,), jnp.float32),
            pltpu.VMEM((L,), jnp.float32),
            pltpu.VMEM((ROWS,), jnp.float32),
            pltpu.SemaphoreType.DMA,
        ],
    )
    u_call = pl.kernel(
        _u_body,
        out_type=jax.ShapeDtypeStruct((B,), jnp.float32),
        mesh=mesh,
        compiler_params=pltpu.CompilerParams(needs_layout_passes=False,
                                             use_tc_tiling_on_sc=True,
                                             skip_device_barrier=True),
        scratch_types=[
            pltpu.VMEM((ROWS,), jnp.int32),
            pltpu.VMEM((ROWS, D), jnp.float32),
            pltpu.VMEM((ROWS * D,), jnp.float32),
            pltpu.VMEM((ROWS,), jnp.float32),
            pltpu.VMEM((ROWS,), jnp.float32),
            pltpu.SemaphoreType.DMA,
        ],
    )
    idx3 = lambda a: a.reshape(NW, NG, GW)
    comb, bsum = pc_call(product_emb, category_emb,
                         user_bias.reshape(-1), product_bias.reshape(-1),
                         global_bias, idx3(user_idx), idx3(product_idx),
                         idx3(category_idx))
    return u_call(user_emb, comb.reshape(-1), bsum, user_idx)
